# Initial kernel scaffold; baseline (speedup 1.0000x reference)
import jax
import jax.numpy as jnp
from jax.experimental import pallas as pl


def kernel(imgs, origin_shapes, cls16, bbox16, param16, cls32, bbox32, param32, pms, u_base, shp_base, exp_base):
    raise NotImplementedError("write your pallas kernel here")


# speedup vs baseline: 10.4464x; 10.4464x over previous
import jax
import jax.numpy as jnp
from jax import lax
from jax.experimental import pallas as pl

_B = 16
_A16 = 800
_A32 = 200
_K = 100
_NEG = -1e9
_MASK = -1e30


_PH = lax.Precision.HIGHEST


def _iota_f32(shape, dim):
    return lax.broadcasted_iota(jnp.int32, shape, dim).astype(jnp.float32)


def _post_kernel(os_ref, ost_ref, cls16_ref, bbox16_ref, param16_ref,
                 cls32_ref, bbox32_ref, param32_ref, pms_ref, u_ref,
                 shpb_ref, expb_ref, acx16_ref, acy16_ref, acx32_ref,
                 acy32_ref, outb_ref, outl_ref):
    f32 = jnp.float32
    ratio = os_ref[...] / 320.0
    ry = ratio[:, 0:1]
    rx = ratio[:, 1:2]
    ratio_t = ost_ref[...] / 320.0
    ry_row = ratio_t[0:1]
    rx_row = ratio_t[1:2]

    w16 = jax.nn.sigmoid(cls16_ref[...])
    w32 = jax.nn.sigmoid(cls32_ref[...])

    acx16 = acx16_ref[...]
    acy16 = acy16_ref[...]
    acx32 = acx32_ref[...]
    acy32 = acy32_ref[...]

    d16 = bbox16_ref[...]
    d32 = bbox32_ref[...]
    x1_16 = (acx16 - d16[0:16] * 16.0) * rx
    y1_16 = (acy16 - d16[16:32] * 16.0) * ry
    x2_16 = (acx16 + d16[32:48] * 16.0) * rx
    y2_16 = (acy16 + d16[48:64] * 16.0) * ry
    x1_32 = (acx32 - d32[0:16] * 32.0) * rx
    y1_32 = (acy32 - d32[16:32] * 32.0) * ry
    x2_32 = (acx32 + d32[32:48] * 32.0) * rx
    y2_32 = (acy32 + d32[48:64] * 32.0) * ry

    w32p = jnp.concatenate(
        [w32, jnp.full((2 * _B, _A16 - _A32), _MASK, f32)], axis=1)
    work0 = jnp.concatenate([w16, w32p], axis=0)

    io_a = _iota_f32((4 * _B, _A16), 1)
    io_k64 = _iota_f32((4 * _B, _K), 1)

    def tk_body(t, st):
        work, vals, idxs = st
        m = jnp.max(work, axis=1, keepdims=True)
        bi = jnp.min(jnp.where(work == m, io_a, 2048.0), axis=1,
                     keepdims=True)
        hit = io_k64 == t.astype(f32)
        vals = jnp.where(hit, m, vals)
        idxs = jnp.where(hit, bi, idxs)
        work = jnp.where(io_a == bi, _MASK, work)
        return work, vals, idxs

    zk = jnp.zeros((4 * _B, _K), f32)
    _, vals64, idx64 = lax.fori_loop(0, _K, tk_body, (work0, zk, zk))

    cand_s = jnp.concatenate([vals64[0:32], vals64[32:64]], axis=1)
    gidx = jnp.concatenate([idx64[0:32], idx64[32:64] + 800.0], axis=1)

    c0_blocks = []
    c1_blocks = []
    for b in range(_B):
        i16 = jnp.concatenate([idx64[b:b + 1], idx64[_B + b:_B + b + 1]],
                              axis=1)
        i32 = jnp.concatenate([idx64[2 * _B + b:2 * _B + b + 1],
                               idx64[3 * _B + b:3 * _B + b + 1]], axis=1)
        oh16 = (_iota_f32((_A16, 2 * _K), 0) == i16).astype(f32)
        oh32 = (_iota_f32((_A32, 2 * _K), 0) == i32).astype(f32)
        bx16 = jnp.concatenate([x1_16[b:b + 1], y1_16[b:b + 1],
                                x2_16[b:b + 1], y2_16[b:b + 1]], axis=0)
        bx32 = jnp.concatenate([x1_32[b:b + 1], y1_32[b:b + 1],
                                x2_32[b:b + 1], y2_32[b:b + 1]], axis=0)
        g16 = jnp.dot(bx16, oh16, precision=_PH)
        g32 = jnp.dot(bx32, oh32, precision=_PH)
        c0_blocks.append(jnp.concatenate([g16[:, 0:_K], g32[:, 0:_K]],
                                         axis=1))
        c1_blocks.append(jnp.concatenate([g16[:, _K:2 * _K],
                                          g32[:, _K:2 * _K]], axis=1))
    cb0 = jnp.stack(c0_blocks, axis=0)
    cb1 = jnp.stack(c1_blocks, axis=0)
    cx1 = jnp.concatenate([cb0[:, 0, :], cb1[:, 0, :]], axis=0)
    cy1 = jnp.concatenate([cb0[:, 1, :], cb1[:, 1, :]], axis=0)
    cx2 = jnp.concatenate([cb0[:, 2, :], cb1[:, 2, :]], axis=0)
    cy2 = jnp.concatenate([cb0[:, 3, :], cb1[:, 3, :]], axis=0)

    worknms = jnp.where(cand_s > 0.5, cand_s, _NEG)
    io_c = _iota_f32((2 * _B, 2 * _K), 1)
    io_k32 = _iota_f32((2 * _B, _K), 1)

    def nms_body(t, st):
        work, ss, sx1, sy1, sx2, sy2, sgi = st
        m = jnp.max(work, axis=1, keepdims=True)
        bi = jnp.min(jnp.where(work == m, io_c, 2048.0), axis=1,
                     keepdims=True)
        sel = (io_c == bi).astype(f32)
        bx1 = jnp.sum(sel * cx1, axis=1, keepdims=True)
        by1 = jnp.sum(sel * cy1, axis=1, keepdims=True)
        bx2 = jnp.sum(sel * cx2, axis=1, keepdims=True)
        by2 = jnp.sum(sel * cy2, axis=1, keepdims=True)
        bgi = jnp.sum(sel * gidx, axis=1, keepdims=True)
        ix1 = jnp.maximum(bx1, cx1)
        iy1 = jnp.maximum(by1, cy1)
        ix2 = jnp.minimum(bx2, cx2)
        iy2 = jnp.minimum(by2, cy2)
        inter = jnp.maximum(ix2 - ix1, 0.0) * jnp.maximum(iy2 - iy1, 0.0)
        aa = jnp.maximum(bx2 - bx1, 0.0) * jnp.maximum(by2 - by1, 0.0)
        ab = jnp.maximum(cx2 - cx1, 0.0) * jnp.maximum(cy2 - cy1, 0.0)
        iou = inter / (aa + ab - inter + 1e-9)
        hit = io_k32 == t.astype(f32)
        ss = jnp.where(hit, m, ss)
        sx1 = jnp.where(hit, bx1, sx1)
        sy1 = jnp.where(hit, by1, sy1)
        sx2 = jnp.where(hit, bx2, sx2)
        sy2 = jnp.where(hit, by2, sy2)
        sgi = jnp.where(hit, bgi, sgi)
        work = jnp.where(iou > 0.45, _NEG, work)
        return work, ss, sx1, sy1, sx2, sy2, sgi

    zk32 = jnp.zeros((2 * _B, _K), f32)
    _, ss, sx1, sy1, sx2, sy2, sgi = lax.fori_loop(
        0, _K, nms_body, (worknms, zk32, zk32, zk32, zk32, zk32, zk32))

    fs = jnp.concatenate([ss[0:_B], ss[_B:2 * _B]], axis=1)
    fx1 = jnp.concatenate([sx1[0:_B], sx1[_B:2 * _B]], axis=1)
    fy1 = jnp.concatenate([sy1[0:_B], sy1[_B:2 * _B]], axis=1)
    fx2 = jnp.concatenate([sx2[0:_B], sx2[_B:2 * _B]], axis=1)
    fy2 = jnp.concatenate([sy2[0:_B], sy2[_B:2 * _B]], axis=1)
    fgi = jnp.concatenate([sgi[0:_B], sgi[_B:2 * _B]], axis=1)
    io_f = _iota_f32((_B, 2 * _K), 1)
    io_k16 = _iota_f32((_B, _K), 1)

    def fin_body(t, st):
        work, tv, ti, ox1, oy1, ox2, oy2, ogi = st
        m = jnp.max(work, axis=1, keepdims=True)
        bi = jnp.min(jnp.where(work == m, io_f, 2048.0), axis=1,
                     keepdims=True)
        sel = (io_f == bi).astype(f32)
        hit = io_k16 == t.astype(f32)
        tv = jnp.where(hit, m, tv)
        ti = jnp.where(hit, bi, ti)
        ox1 = jnp.where(hit, jnp.sum(sel * fx1, axis=1, keepdims=True), ox1)
        oy1 = jnp.where(hit, jnp.sum(sel * fy1, axis=1, keepdims=True), oy1)
        ox2 = jnp.where(hit, jnp.sum(sel * fx2, axis=1, keepdims=True), ox2)
        oy2 = jnp.where(hit, jnp.sum(sel * fy2, axis=1, keepdims=True), oy2)
        ogi = jnp.where(hit, jnp.sum(sel * fgi, axis=1, keepdims=True), ogi)
        work = jnp.where(io_f == bi, _MASK, work)
        return work, tv, ti, ox1, oy1, ox2, oy2, ogi

    zk16 = jnp.zeros((_B, _K), f32)
    _, tv, ti, ox1, oy1, ox2, oy2, ogi = lax.fori_loop(
        0, _K, fin_body, (fs, zk16, zk16, zk16, zk16, zk16, zk16, zk16))

    oc = jnp.floor(ti / 100.0)
    outb_ref[:, :, 0] = ox1
    outb_ref[:, :, 1] = oy1
    outb_ref[:, :, 2] = ox2
    outb_ref[:, :, 3] = oy2
    outb_ref[:, :, 4] = tv
    outb_ref[:, :, 5] = oc

    plist = []
    for b in range(_B):
        gi_b = ogi[b:b + 1]
        oh16 = (_iota_f32((_A16, _K), 0) == gi_b).astype(f32)
        oh32 = (_iota_f32((_A32, _K), 0) == (gi_b - 800.0)
                ).astype(f32)
        pb = (lax.dot_general(oh16, param16_ref[b],
                              (((0,), (0,)), ((), ())), precision=_PH) +
              lax.dot_general(oh32, param32_ref[b],
                              (((0,), (0,)), ((), ())), precision=_PH))
        plist.append(pb)
    sel_p = jnp.concatenate(plist, axis=0)

    pms = pms_ref[...]
    p = sel_p * pms[1:2, :] + pms[0:1, :]
    p9 = p[:, 0:9]
    shp = p[:, 9:208]
    expc = p[:, 208:237]
    sv = lax.dot_general(shp, shpb_ref[...], (((1,), (1,)), ((), ())), precision=_PH)
    ev = lax.dot_general(expc, expb_ref[...], (((1,), (1,)), ((), ())), precision=_PH)
    v = u_ref[...] + sv + ev

    io_r = _iota_f32((204, 68), 0)
    io_k68 = _iota_f32((204, 68), 1)
    vx = jnp.dot(v, (io_r == io_k68 * 3.0).astype(f32), precision=_PH)
    vy = jnp.dot(v, (io_r == io_k68 * 3.0 + 1.0).astype(f32), precision=_PH)
    vz = jnp.dot(v, (io_r == io_k68 * 3.0 + 2.0).astype(f32), precision=_PH)

    lx = vx * p9[:, 0:1] + vy * p9[:, 1:2] + vz * p9[:, 2:3]
    ly = vx * p9[:, 3:4] + vy * p9[:, 4:5] + vz * p9[:, 5:6]

    rep = (lax.broadcasted_iota(jnp.int32, (_B * _K, _B), 0) // _K
           == lax.broadcasted_iota(jnp.int32, (_B * _K, _B), 1)
           ).astype(f32)
    rxn = jnp.sum(rep * rx_row, axis=1, keepdims=True)
    ryn = jnp.sum(rep * ry_row, axis=1, keepdims=True)
    lxs = lx * rxn
    lys = ly * ryn

    io_68r = _iota_f32((68, 136), 0)
    io_136 = _iota_f32((68, 136), 1)
    sx_m = (io_136 == io_68r * 2.0).astype(f32)
    sy_m = (io_136 == io_68r * 2.0 + 1.0).astype(f32)
    outl_ref[...] = (jnp.dot(lxs, sx_m, precision=_PH) +
                     jnp.dot(lys, sy_m, precision=_PH))


def _anchor_xy(stride):
    hw = 320 // stride
    X, Y = jnp.meshgrid(jnp.arange(hw), jnp.arange(hw))
    ac = jnp.stack([X, Y], axis=-1).reshape(-1, 2) * stride
    ac = jnp.stack([ac, ac], axis=1).reshape(-1, 2).astype(jnp.float32)
    return ac[:, 0].reshape(1, -1), ac[:, 1].reshape(1, -1)


def kernel(imgs, origin_shapes, cls16, bbox16, param16, cls32, bbox32,
           param32, pms, u_base, shp_base, exp_base):
    del imgs
    acx16, acy16 = _anchor_xy(16)
    acx32, acy32 = _anchor_xy(32)
    u204 = u_base.reshape(1, 204)
    os_t = origin_shapes.T
    cls16t = cls16.transpose(2, 0, 1).reshape(2 * _B, _A16)
    cls32t = cls32.transpose(2, 0, 1).reshape(2 * _B, _A32)
    bbox16t = bbox16.transpose(2, 0, 1).reshape(4 * _B, _A16)
    bbox32t = bbox32.transpose(2, 0, 1).reshape(4 * _B, _A32)
    out_shape = (
        jax.ShapeDtypeStruct((_B, _K, 6), jnp.float32),
        jax.ShapeDtypeStruct((_B * _K, 2 * 68), jnp.float32),
    )
    bb6, lmk = pl.pallas_call(_post_kernel, out_shape=out_shape)(
        origin_shapes, os_t, cls16t, bbox16t, param16, cls32t, bbox32t,
        param32, pms, u204, shp_base, exp_base, acx16, acy16, acx32, acy32)
    return bb6, lmk.reshape(_B, _K, 68, 2)


# speedup vs baseline: 12.4841x; 1.1951x over previous
import jax
import jax.numpy as jnp
from jax import lax
from jax.experimental import pallas as pl

_B = 16
_A16 = 800
_A32 = 200
_K = 100
_NEG = -1e9
_MASK = -1e30


_PH = lax.Precision.HIGHEST
_P3 = lax.Precision.HIGHEST


def _iota_f32(shape, dim):
    return lax.broadcasted_iota(jnp.int32, shape, dim).astype(jnp.float32)


def _post_kernel(os_ref, ost_ref, cls16_ref, bbox16_ref, param16_ref,
                 cls32_ref, bbox32_ref, param32_ref, pms_ref, u_ref,
                 shpb_ref, expb_ref, acx16_ref, acy16_ref, acx32_ref,
                 acy32_ref, outb_ref, outl_ref):
    f32 = jnp.float32
    ratio = os_ref[...] / 320.0
    ry = ratio[:, 0:1]
    rx = ratio[:, 1:2]
    ratio_t = ost_ref[...] / 320.0
    ry_row = ratio_t[0:1]
    rx_row = ratio_t[1:2]

    w16 = jax.nn.sigmoid(cls16_ref[...])
    w32 = jax.nn.sigmoid(cls32_ref[...])

    acx16 = acx16_ref[...]
    acy16 = acy16_ref[...]
    acx32 = acx32_ref[...]
    acy32 = acy32_ref[...]

    d16 = bbox16_ref[...]
    d32 = bbox32_ref[...]
    x1_16 = (acx16 - d16[0:16] * 16.0) * rx
    y1_16 = (acy16 - d16[16:32] * 16.0) * ry
    x2_16 = (acx16 + d16[32:48] * 16.0) * rx
    y2_16 = (acy16 + d16[48:64] * 16.0) * ry
    x1_32 = (acx32 - d32[0:16] * 32.0) * rx
    y1_32 = (acy32 - d32[16:32] * 32.0) * ry
    x2_32 = (acx32 + d32[32:48] * 32.0) * rx
    y2_32 = (acy32 + d32[48:64] * 32.0) * ry

    w32p = jnp.concatenate(
        [w32, jnp.full((2 * _B, _A16 - _A32), _MASK, f32)], axis=1)
    work0 = jnp.concatenate([w16, w32p], axis=0)

    io_a = _iota_f32((4 * _B, _A16), 1)
    io_k64 = _iota_f32((4 * _B, _K), 1)

    def tk_body(t, st):
        work, vals, idxs = st
        m = jnp.max(work, axis=1, keepdims=True)
        bi = jnp.min(jnp.where(work == m, io_a, 2048.0), axis=1,
                     keepdims=True)
        hit = io_k64 == t.astype(f32)
        vals = jnp.where(hit, m, vals)
        idxs = jnp.where(hit, bi, idxs)
        work = jnp.where(io_a == bi, _MASK, work)
        return work, vals, idxs

    zk = jnp.zeros((4 * _B, _K), f32)
    _, vals64, idx64 = lax.fori_loop(0, _K, tk_body, (work0, zk, zk))

    cand_s = jnp.concatenate([vals64[0:32], vals64[32:64]], axis=1)
    gidx = jnp.concatenate([idx64[0:32], idx64[32:64] + 800.0], axis=1)

    c0_blocks = []
    c1_blocks = []
    for b in range(_B):
        i16 = jnp.concatenate([idx64[b:b + 1], idx64[_B + b:_B + b + 1]],
                              axis=1)
        i32 = jnp.concatenate([idx64[2 * _B + b:2 * _B + b + 1],
                               idx64[3 * _B + b:3 * _B + b + 1]], axis=1)
        oh16 = (_iota_f32((_A16, 2 * _K), 0) == i16).astype(f32)
        oh32 = (_iota_f32((_A32, 2 * _K), 0) == i32).astype(f32)
        bx16 = jnp.concatenate([x1_16[b:b + 1], y1_16[b:b + 1],
                                x2_16[b:b + 1], y2_16[b:b + 1]], axis=0)
        bx32 = jnp.concatenate([x1_32[b:b + 1], y1_32[b:b + 1],
                                x2_32[b:b + 1], y2_32[b:b + 1]], axis=0)
        g16 = jnp.dot(bx16, oh16, precision=_PH)
        g32 = jnp.dot(bx32, oh32, precision=_PH)
        c0_blocks.append(jnp.concatenate([g16[:, 0:_K], g32[:, 0:_K]],
                                         axis=1))
        c1_blocks.append(jnp.concatenate([g16[:, _K:2 * _K],
                                          g32[:, _K:2 * _K]], axis=1))
    cb0 = jnp.stack(c0_blocks, axis=0)
    cb1 = jnp.stack(c1_blocks, axis=0)
    cx1 = jnp.concatenate([cb0[:, 0, :], cb1[:, 0, :]], axis=0)
    cy1 = jnp.concatenate([cb0[:, 1, :], cb1[:, 1, :]], axis=0)
    cx2 = jnp.concatenate([cb0[:, 2, :], cb1[:, 2, :]], axis=0)
    cy2 = jnp.concatenate([cb0[:, 3, :], cb1[:, 3, :]], axis=0)

    worknms = jnp.where(cand_s > 0.5, cand_s, _NEG)
    io_c = _iota_f32((2 * _B, 2 * _K), 1)
    io_k32 = _iota_f32((2 * _B, _K), 1)

    def nms_body(t, st):
        work, ss, sx1, sy1, sx2, sy2, sgi = st
        m = jnp.max(work, axis=1, keepdims=True)
        bi = jnp.min(jnp.where(work == m, io_c, 2048.0), axis=1,
                     keepdims=True)
        sel = (io_c == bi).astype(f32)
        bx1 = jnp.sum(sel * cx1, axis=1, keepdims=True)
        by1 = jnp.sum(sel * cy1, axis=1, keepdims=True)
        bx2 = jnp.sum(sel * cx2, axis=1, keepdims=True)
        by2 = jnp.sum(sel * cy2, axis=1, keepdims=True)
        bgi = jnp.sum(sel * gidx, axis=1, keepdims=True)
        ix1 = jnp.maximum(bx1, cx1)
        iy1 = jnp.maximum(by1, cy1)
        ix2 = jnp.minimum(bx2, cx2)
        iy2 = jnp.minimum(by2, cy2)
        inter = jnp.maximum(ix2 - ix1, 0.0) * jnp.maximum(iy2 - iy1, 0.0)
        aa = jnp.maximum(bx2 - bx1, 0.0) * jnp.maximum(by2 - by1, 0.0)
        ab = jnp.maximum(cx2 - cx1, 0.0) * jnp.maximum(cy2 - cy1, 0.0)
        iou = inter / (aa + ab - inter + 1e-9)
        hit = io_k32 == t.astype(f32)
        ss = jnp.where(hit, m, ss)
        sx1 = jnp.where(hit, bx1, sx1)
        sy1 = jnp.where(hit, by1, sy1)
        sx2 = jnp.where(hit, bx2, sx2)
        sy2 = jnp.where(hit, by2, sy2)
        sgi = jnp.where(hit, bgi, sgi)
        work = jnp.where(iou > 0.45, _NEG, work)
        return work, ss, sx1, sy1, sx2, sy2, sgi

    zk32 = jnp.zeros((2 * _B, _K), f32)
    _, ss, sx1, sy1, sx2, sy2, sgi = lax.fori_loop(
        0, _K, nms_body, (worknms, zk32, zk32, zk32, zk32, zk32, zk32))

    fs = jnp.concatenate([ss[0:_B], ss[_B:2 * _B]], axis=1)
    fx1 = jnp.concatenate([sx1[0:_B], sx1[_B:2 * _B]], axis=1)
    fy1 = jnp.concatenate([sy1[0:_B], sy1[_B:2 * _B]], axis=1)
    fx2 = jnp.concatenate([sx2[0:_B], sx2[_B:2 * _B]], axis=1)
    fy2 = jnp.concatenate([sy2[0:_B], sy2[_B:2 * _B]], axis=1)
    fgi = jnp.concatenate([sgi[0:_B], sgi[_B:2 * _B]], axis=1)
    fsT = jnp.transpose(fs)

    io_sub = _iota_f32((2 * _K, 2 * _K), 0)
    io_lan = _iota_f32((2 * _K, 2 * _K), 1)
    io_rk = _iota_f32((2 * _K, _K), 1)
    io_idx = _iota_f32((1, 2 * _K), 1)
    blks = []
    for b in range(_B):
        a_row = fs[b:b + 1]
        a_col = fsT[:, b:b + 1]
        gt = a_row > a_col
        tie = (a_row == a_col) & (io_lan < io_sub)
        rank_col = jnp.sum((gt | tie).astype(f32), axis=1,
                           keepdims=True)
        ohrt = (io_rk == rank_col).astype(f32)
        pay = jnp.concatenate([fx1[b:b + 1], fy1[b:b + 1], fx2[b:b + 1],
                               fy2[b:b + 1], fs[b:b + 1], fgi[b:b + 1],
                               io_idx], axis=0)
        blks.append(jnp.dot(pay, ohrt, precision=_P3))
    blk3 = jnp.stack(blks, axis=0)
    ox1 = blk3[:, 0, :]
    oy1 = blk3[:, 1, :]
    ox2 = blk3[:, 2, :]
    oy2 = blk3[:, 3, :]
    tv = blk3[:, 4, :]
    ogi = jnp.floor(blk3[:, 5, :] + 0.5)
    ti = jnp.floor(blk3[:, 6, :] + 0.5)

    oc = jnp.floor(ti / 100.0)
    outb_ref[:, :, 0] = ox1
    outb_ref[:, :, 1] = oy1
    outb_ref[:, :, 2] = ox2
    outb_ref[:, :, 3] = oy2
    outb_ref[:, :, 4] = tv
    outb_ref[:, :, 5] = oc

    plist = []
    for b in range(_B):
        gi_b = ogi[b:b + 1]
        oh16 = (_iota_f32((_A16, _K), 0) == gi_b).astype(f32)
        oh32 = (_iota_f32((_A32, _K), 0) == (gi_b - 800.0)
                ).astype(f32)
        pb = (lax.dot_general(oh16, param16_ref[b],
                              (((0,), (0,)), ((), ())), precision=_P3) +
              lax.dot_general(oh32, param32_ref[b],
                              (((0,), (0,)), ((), ())), precision=_P3))
        plist.append(pb)
    sel_p = jnp.concatenate(plist, axis=0)

    pms = pms_ref[...]
    p = sel_p * pms[1:2, :] + pms[0:1, :]
    p9 = p[:, 0:9]
    shp = p[:, 9:208]
    expc = p[:, 208:237]
    sv = lax.dot_general(shp, shpb_ref[...], (((1,), (1,)), ((), ())), precision=_P3)
    ev = lax.dot_general(expc, expb_ref[...], (((1,), (1,)), ((), ())), precision=_P3)
    v = u_ref[...] + sv + ev

    io_r = _iota_f32((204, 68), 0)
    io_k68 = _iota_f32((204, 68), 1)
    vx = jnp.dot(v, (io_r == io_k68 * 3.0).astype(f32), precision=_P3)
    vy = jnp.dot(v, (io_r == io_k68 * 3.0 + 1.0).astype(f32), precision=_P3)
    vz = jnp.dot(v, (io_r == io_k68 * 3.0 + 2.0).astype(f32), precision=_P3)

    lx = vx * p9[:, 0:1] + vy * p9[:, 1:2] + vz * p9[:, 2:3]
    ly = vx * p9[:, 3:4] + vy * p9[:, 4:5] + vz * p9[:, 5:6]

    rep = (lax.broadcasted_iota(jnp.int32, (_B * _K, _B), 0) // _K
           == lax.broadcasted_iota(jnp.int32, (_B * _K, _B), 1)
           ).astype(f32)
    rxn = jnp.sum(rep * rx_row, axis=1, keepdims=True)
    ryn = jnp.sum(rep * ry_row, axis=1, keepdims=True)
    lxs = lx * rxn
    lys = ly * ryn

    io_68r = _iota_f32((68, 136), 0)
    io_136 = _iota_f32((68, 136), 1)
    sx_m = (io_136 == io_68r * 2.0).astype(f32)
    sy_m = (io_136 == io_68r * 2.0 + 1.0).astype(f32)
    outl_ref[...] = (jnp.dot(lxs, sx_m, precision=_P3) +
                     jnp.dot(lys, sy_m, precision=_P3))


def _anchor_xy(stride):
    hw = 320 // stride
    X, Y = jnp.meshgrid(jnp.arange(hw), jnp.arange(hw))
    ac = jnp.stack([X, Y], axis=-1).reshape(-1, 2) * stride
    ac = jnp.stack([ac, ac], axis=1).reshape(-1, 2).astype(jnp.float32)
    return ac[:, 0].reshape(1, -1), ac[:, 1].reshape(1, -1)


def kernel(imgs, origin_shapes, cls16, bbox16, param16, cls32, bbox32,
           param32, pms, u_base, shp_base, exp_base):
    del imgs
    acx16, acy16 = _anchor_xy(16)
    acx32, acy32 = _anchor_xy(32)
    u204 = u_base.reshape(1, 204)
    os_t = origin_shapes.T
    cls16t = cls16.transpose(2, 0, 1).reshape(2 * _B, _A16)
    cls32t = cls32.transpose(2, 0, 1).reshape(2 * _B, _A32)
    bbox16t = bbox16.transpose(2, 0, 1).reshape(4 * _B, _A16)
    bbox32t = bbox32.transpose(2, 0, 1).reshape(4 * _B, _A32)
    out_shape = (
        jax.ShapeDtypeStruct((_B, _K, 6), jnp.float32),
        jax.ShapeDtypeStruct((_B * _K, 2 * 68), jnp.float32),
    )
    bb6, lmk = pl.pallas_call(_post_kernel, out_shape=out_shape)(
        origin_shapes, os_t, cls16t, bbox16t, param16, cls32t, bbox32t,
        param32, pms, u204, shp_base, exp_base, acx16, acy16, acx32, acy32)
    return bb6, lmk.reshape(_B, _K, 68, 2)


# speedup vs baseline: 12.5009x; 1.0013x over previous
import jax
import jax.numpy as jnp
from jax import lax
from jax.experimental import pallas as pl

_B = 16
_A16 = 800
_A32 = 200
_K = 100
_NEG = -1e9
_MASK = -1e30


_PH = lax.Precision.HIGHEST
_P3 = lax.Precision.HIGHEST


def _iota_f32(shape, dim):
    return lax.broadcasted_iota(jnp.int32, shape, dim).astype(jnp.float32)


def _post_kernel(os_ref, ost_ref, cls16_ref, bbox16_ref, param16_ref,
                 cls32_ref, bbox32_ref, param32_ref, pms_ref, u_ref,
                 shpb_ref, expb_ref, acx16_ref, acy16_ref, acx32_ref,
                 acy32_ref, outb_ref, outlx_ref, outly_ref):
    f32 = jnp.float32
    ratio = os_ref[...] / 320.0
    ry = ratio[:, 0:1]
    rx = ratio[:, 1:2]
    ratio_t = ost_ref[...] / 320.0
    ry_row = ratio_t[0:1]
    rx_row = ratio_t[1:2]

    w16 = jax.nn.sigmoid(cls16_ref[...])
    w32 = jax.nn.sigmoid(cls32_ref[...])

    acx16 = acx16_ref[...]
    acy16 = acy16_ref[...]
    acx32 = acx32_ref[...]
    acy32 = acy32_ref[...]

    d16 = bbox16_ref[...]
    d32 = bbox32_ref[...]
    x1_16 = (acx16 - d16[0:16] * 16.0) * rx
    y1_16 = (acy16 - d16[16:32] * 16.0) * ry
    x2_16 = (acx16 + d16[32:48] * 16.0) * rx
    y2_16 = (acy16 + d16[48:64] * 16.0) * ry
    x1_32 = (acx32 - d32[0:16] * 32.0) * rx
    y1_32 = (acy32 - d32[16:32] * 32.0) * ry
    x2_32 = (acx32 + d32[32:48] * 32.0) * rx
    y2_32 = (acy32 + d32[48:64] * 32.0) * ry

    w32p = jnp.concatenate(
        [w32, jnp.full((2 * _B, _A16 - _A32), _MASK, f32)], axis=1)
    work0 = jnp.concatenate([w16, w32p], axis=0)

    io_a = _iota_f32((4 * _B, _A16), 1)
    io_k64 = _iota_f32((4 * _B, _K), 1)

    def tk_body(t, st):
        work, vals, idxs = st
        m = jnp.max(work, axis=1, keepdims=True)
        bi = jnp.min(jnp.where(work == m, io_a, 2048.0), axis=1,
                     keepdims=True)
        hit = io_k64 == t.astype(f32)
        vals = jnp.where(hit, m, vals)
        idxs = jnp.where(hit, bi, idxs)
        work = jnp.where(io_a == bi, _MASK, work)
        return work, vals, idxs

    zk = jnp.zeros((4 * _B, _K), f32)
    _, vals64, idx64 = lax.fori_loop(0, _K, tk_body, (work0, zk, zk))

    cand_s = jnp.concatenate([vals64[0:32], vals64[32:64]], axis=1)
    gidx = jnp.concatenate([idx64[0:32], idx64[32:64] + 800.0], axis=1)

    c0_blocks = []
    c1_blocks = []
    for b in range(_B):
        i16 = jnp.concatenate([idx64[b:b + 1], idx64[_B + b:_B + b + 1]],
                              axis=1)
        i32 = jnp.concatenate([idx64[2 * _B + b:2 * _B + b + 1],
                               idx64[3 * _B + b:3 * _B + b + 1]], axis=1)
        oh16 = (_iota_f32((_A16, 2 * _K), 0) == i16).astype(f32)
        oh32 = (_iota_f32((_A32, 2 * _K), 0) == i32).astype(f32)
        bx16 = jnp.concatenate([x1_16[b:b + 1], y1_16[b:b + 1],
                                x2_16[b:b + 1], y2_16[b:b + 1]], axis=0)
        bx32 = jnp.concatenate([x1_32[b:b + 1], y1_32[b:b + 1],
                                x2_32[b:b + 1], y2_32[b:b + 1]], axis=0)
        g16 = jnp.dot(bx16, oh16, precision=_PH)
        g32 = jnp.dot(bx32, oh32, precision=_PH)
        c0_blocks.append(jnp.concatenate([g16[:, 0:_K], g32[:, 0:_K]],
                                         axis=1))
        c1_blocks.append(jnp.concatenate([g16[:, _K:2 * _K],
                                          g32[:, _K:2 * _K]], axis=1))
    cb0 = jnp.stack(c0_blocks, axis=0)
    cb1 = jnp.stack(c1_blocks, axis=0)
    cx1 = jnp.concatenate([cb0[:, 0, :], cb1[:, 0, :]], axis=0)
    cy1 = jnp.concatenate([cb0[:, 1, :], cb1[:, 1, :]], axis=0)
    cx2 = jnp.concatenate([cb0[:, 2, :], cb1[:, 2, :]], axis=0)
    cy2 = jnp.concatenate([cb0[:, 3, :], cb1[:, 3, :]], axis=0)

    worknms = jnp.where(cand_s > 0.5, cand_s, _NEG)
    io_c = _iota_f32((2 * _B, 2 * _K), 1)
    io_k32 = _iota_f32((2 * _B, _K), 1)

    def nms_body(t, st):
        work, ss, sx1, sy1, sx2, sy2, sgi = st
        m = jnp.max(work, axis=1, keepdims=True)
        bi = jnp.min(jnp.where(work == m, io_c, 2048.0), axis=1,
                     keepdims=True)
        sel = (io_c == bi).astype(f32)
        bx1 = jnp.sum(sel * cx1, axis=1, keepdims=True)
        by1 = jnp.sum(sel * cy1, axis=1, keepdims=True)
        bx2 = jnp.sum(sel * cx2, axis=1, keepdims=True)
        by2 = jnp.sum(sel * cy2, axis=1, keepdims=True)
        bgi = jnp.sum(sel * gidx, axis=1, keepdims=True)
        ix1 = jnp.maximum(bx1, cx1)
        iy1 = jnp.maximum(by1, cy1)
        ix2 = jnp.minimum(bx2, cx2)
        iy2 = jnp.minimum(by2, cy2)
        inter = jnp.maximum(ix2 - ix1, 0.0) * jnp.maximum(iy2 - iy1, 0.0)
        aa = jnp.maximum(bx2 - bx1, 0.0) * jnp.maximum(by2 - by1, 0.0)
        ab = jnp.maximum(cx2 - cx1, 0.0) * jnp.maximum(cy2 - cy1, 0.0)
        iou = inter / (aa + ab - inter + 1e-9)
        hit = io_k32 == t.astype(f32)
        ss = jnp.where(hit, m, ss)
        sx1 = jnp.where(hit, bx1, sx1)
        sy1 = jnp.where(hit, by1, sy1)
        sx2 = jnp.where(hit, bx2, sx2)
        sy2 = jnp.where(hit, by2, sy2)
        sgi = jnp.where(hit, bgi, sgi)
        work = jnp.where(iou > 0.45, _NEG, work)
        return work, ss, sx1, sy1, sx2, sy2, sgi

    zk32 = jnp.zeros((2 * _B, _K), f32)
    _, ss, sx1, sy1, sx2, sy2, sgi = lax.fori_loop(
        0, _K, nms_body, (worknms, zk32, zk32, zk32, zk32, zk32, zk32))

    fs = jnp.concatenate([ss[0:_B], ss[_B:2 * _B]], axis=1)
    fx1 = jnp.concatenate([sx1[0:_B], sx1[_B:2 * _B]], axis=1)
    fy1 = jnp.concatenate([sy1[0:_B], sy1[_B:2 * _B]], axis=1)
    fx2 = jnp.concatenate([sx2[0:_B], sx2[_B:2 * _B]], axis=1)
    fy2 = jnp.concatenate([sy2[0:_B], sy2[_B:2 * _B]], axis=1)
    fgi = jnp.concatenate([sgi[0:_B], sgi[_B:2 * _B]], axis=1)
    fsT = jnp.transpose(fs)

    io_sub = _iota_f32((2 * _K, 2 * _K), 0)
    io_lan = _iota_f32((2 * _K, 2 * _K), 1)
    io_rk = _iota_f32((2 * _K, _K), 1)
    io_idx = _iota_f32((1, 2 * _K), 1)
    blks = []
    for b in range(_B):
        a_row = fs[b:b + 1]
        a_col = fsT[:, b:b + 1]
        gt = a_row > a_col
        tie = (a_row == a_col) & (io_lan < io_sub)
        rank_col = jnp.sum((gt | tie).astype(f32), axis=1,
                           keepdims=True)
        ohrt = (io_rk == rank_col).astype(f32)
        pay = jnp.concatenate([fx1[b:b + 1], fy1[b:b + 1], fx2[b:b + 1],
                               fy2[b:b + 1], fs[b:b + 1], fgi[b:b + 1],
                               io_idx], axis=0)
        blks.append(jnp.dot(pay, ohrt, precision=_P3))
    blk3 = jnp.stack(blks, axis=0)
    ox1 = blk3[:, 0, :]
    oy1 = blk3[:, 1, :]
    ox2 = blk3[:, 2, :]
    oy2 = blk3[:, 3, :]
    tv = blk3[:, 4, :]
    ogi = jnp.floor(blk3[:, 5, :] + 0.5)
    ti = jnp.floor(blk3[:, 6, :] + 0.5)

    oc = jnp.floor(ti / 100.0)
    outb_ref[:, :, 0] = ox1
    outb_ref[:, :, 1] = oy1
    outb_ref[:, :, 2] = ox2
    outb_ref[:, :, 3] = oy2
    outb_ref[:, :, 4] = tv
    outb_ref[:, :, 5] = oc

    plist = []
    for b in range(_B):
        gi_b = ogi[b:b + 1]
        oh16 = (_iota_f32((_A16, _K), 0) == gi_b).astype(f32)
        oh32 = (_iota_f32((_A32, _K), 0) == (gi_b - 800.0)
                ).astype(f32)
        pb = (lax.dot_general(oh16, param16_ref[b],
                              (((0,), (0,)), ((), ())), precision=_P3) +
              lax.dot_general(oh32, param32_ref[b],
                              (((0,), (0,)), ((), ())), precision=_P3))
        plist.append(pb)
    sel_p = jnp.concatenate(plist, axis=0)

    pms = pms_ref[...]
    p = sel_p * pms[1:2, :] + pms[0:1, :]
    p9 = p[:, 0:9]
    shp = p[:, 9:208]
    expc = p[:, 208:237]
    shpb = shpb_ref[...]
    expb = expb_ref[...]
    u3 = u_ref[...]
    vx = (u3[0:1] +
          lax.dot_general(shp, shpb[0:68], (((1,), (1,)), ((), ())),
                          precision=_P3) +
          lax.dot_general(expc, expb[0:68], (((1,), (1,)), ((), ())),
                          precision=_P3))
    vy = (u3[1:2] +
          lax.dot_general(shp, shpb[68:136], (((1,), (1,)), ((), ())),
                          precision=_P3) +
          lax.dot_general(expc, expb[68:136], (((1,), (1,)), ((), ())),
                          precision=_P3))
    vz = (u3[2:3] +
          lax.dot_general(shp, shpb[136:204], (((1,), (1,)), ((), ())),
                          precision=_P3) +
          lax.dot_general(expc, expb[136:204], (((1,), (1,)), ((), ())),
                          precision=_P3))

    lx = vx * p9[:, 0:1] + vy * p9[:, 1:2] + vz * p9[:, 2:3]
    ly = vx * p9[:, 3:4] + vy * p9[:, 4:5] + vz * p9[:, 5:6]

    rep = (lax.broadcasted_iota(jnp.int32, (_B * _K, _B), 0) // _K
           == lax.broadcasted_iota(jnp.int32, (_B * _K, _B), 1)
           ).astype(f32)
    rxn = jnp.sum(rep * rx_row, axis=1, keepdims=True)
    ryn = jnp.sum(rep * ry_row, axis=1, keepdims=True)
    outlx_ref[...] = lx * rxn
    outly_ref[...] = ly * ryn


def _anchor_xy(stride):
    hw = 320 // stride
    X, Y = jnp.meshgrid(jnp.arange(hw), jnp.arange(hw))
    ac = jnp.stack([X, Y], axis=-1).reshape(-1, 2) * stride
    ac = jnp.stack([ac, ac], axis=1).reshape(-1, 2).astype(jnp.float32)
    return ac[:, 0].reshape(1, -1), ac[:, 1].reshape(1, -1)


def kernel(imgs, origin_shapes, cls16, bbox16, param16, cls32, bbox32,
           param32, pms, u_base, shp_base, exp_base):
    del imgs
    acx16, acy16 = _anchor_xy(16)
    acx32, acy32 = _anchor_xy(32)
    u3 = u_base.reshape(68, 3).T
    shpb_s = jnp.concatenate([shp_base[0::3], shp_base[1::3],
                              shp_base[2::3]], axis=0)
    expb_s = jnp.concatenate([exp_base[0::3], exp_base[1::3],
                              exp_base[2::3]], axis=0)
    os_t = origin_shapes.T
    cls16t = cls16.transpose(2, 0, 1).reshape(2 * _B, _A16)
    cls32t = cls32.transpose(2, 0, 1).reshape(2 * _B, _A32)
    bbox16t = bbox16.transpose(2, 0, 1).reshape(4 * _B, _A16)
    bbox32t = bbox32.transpose(2, 0, 1).reshape(4 * _B, _A32)
    out_shape = (
        jax.ShapeDtypeStruct((_B, _K, 6), jnp.float32),
        jax.ShapeDtypeStruct((_B * _K, 68), jnp.float32),
        jax.ShapeDtypeStruct((_B * _K, 68), jnp.float32),
    )
    bb6, lmx, lmy = pl.pallas_call(_post_kernel, out_shape=out_shape)(
        origin_shapes, os_t, cls16t, bbox16t, param16, cls32t, bbox32t,
        param32, pms, u3, shpb_s, expb_s, acx16, acy16, acx32, acy32)
    return bb6, jnp.stack([lmx, lmy], axis=-1).reshape(_B, _K, 68, 2)
